# baseline (speedup 1.0000x reference)
import functools

import jax
import jax.numpy as jnp
from jax import lax
from jax.experimental import pallas as pl
from jax.experimental.pallas import tpu as pltpu
from jax.experimental.pallas import tpu_sc as plsc

_A = 1280
_FR = 0.15
_TR = 0.2

_NC = 2
_NS = 16
_L = 16
_CH = 8
_NBUF = 4


def _mask_bounds(lengths):
    B = lengths.shape[0]
    Ti = lengths.astype(jnp.int32)
    key = jax.random.key(42)
    key, ka, kb = jax.random.split(key, 3)
    max_t = jnp.maximum(1, jnp.floor(Ti.astype(jnp.float32) * _TR).astype(jnp.int32))
    u = jax.random.uniform(ka, (B,))
    t = 1 + jnp.floor(u * max_t.astype(jnp.float32)).astype(jnp.int32)
    t = jnp.minimum(t, max_t)
    room = Ti - t
    u2 = jax.random.uniform(kb, (B,))
    t0 = jnp.where(room > 0,
                   jnp.floor(u2 * (room + 1).astype(jnp.float32)).astype(jnp.int32),
                   0)
    valid = Ti > 0
    big = jnp.int32(1 << 30)
    tlo = jnp.where(valid, t0, big)
    thi = jnp.where(valid, t0 + t, big)
    max_f = int(_A * _FR)
    key, ka2, kb2 = jax.random.split(key, 3)
    f = jax.random.randint(ka2, (B,), 1, max_f + 1)
    f0_max = jnp.clip(_A - f, 0, None)
    f0 = jnp.floor(jax.random.uniform(kb2, (B,)) * (f0_max + 1).astype(jnp.float32)
                   ).astype(jnp.int32)
    return tlo, thi, f0, f0 + f


def _sc_call(X, params):
    B, T, D = X.shape
    off = D - _A
    nw = _NC * _NS
    rows_w = (B * T) // nw
    wps = T // rows_w
    n_chunks = rows_w // _CH
    mesh = plsc.VectorSubcoreMesh(core_axis_name="c", subcore_axis_name="s")

    n_rounds = n_chunks // _NBUF

    @functools.partial(
        pl.kernel,
        mesh=mesh,
        out_type=jax.ShapeDtypeStruct((B, T, D), X.dtype),
        scratch_types=(
            [pltpu.VMEM((_CH, _A), jnp.float32)] * _NBUF
            + [pltpu.VMEM((4, _L), jnp.int32)]
            + [pltpu.SemaphoreType.DMA] * (3 * _NBUF)
        ),
    )
    def k(x_hbm, p_hbm, out_hbm, *rest):
        bufs = rest[:_NBUF]
        pv = rest[_NBUF]
        isems = rest[_NBUF + 1:_NBUF + 1 + _NBUF]
        osems = rest[_NBUF + 1 + _NBUF:_NBUF + 1 + 2 * _NBUF]
        vsems = rest[_NBUF + 1 + 2 * _NBUF:]
        cid = lax.axis_index("c")
        sid = lax.axis_index("s")
        wid = sid * _NC + cid
        b = wid // wps
        row0 = (wid % wps) * rows_w
        pltpu.sync_copy(p_hbm.at[b], pv)
        tlo = pv[0][0]
        thi = pv[1][0]
        flo = pv[2][0]
        fhi = pv[3][0]

        def in_desc(ci, s):
            r = row0 + ci * _CH
            return pltpu.make_async_copy(
                x_hbm.at[b, pl.ds(r, _CH), pl.ds(off, _A)], bufs[s], isems[s])

        def out_desc(ci, s):
            r = row0 + ci * _CH
            return pltpu.make_async_copy(
                bufs[s], out_hbm.at[b, pl.ds(r, _CH), pl.ds(off, _A)], osems[s])

        def v_desc():
            return pltpu.make_async_copy(
                x_hbm.at[b, pl.ds(row0, rows_w), pl.ds(0, off)],
                out_hbm.at[b, pl.ds(row0, rows_w), pl.ds(0, off)], vsems[0])

        flo_l = flo - off
        fhi_l = fhi - off
        n_fg = int(_A * _FR) // _L + 2
        fg0 = flo_l // _L
        lanes = lax.iota(jnp.int32, _L)
        fcols = []
        fmasks = []
        for u in range(n_fg):
            g = jnp.minimum(fg0 + u, (_A // _L) - 1)
            c0 = g * _L
            col = c0 + lanes
            fcols.append(c0)
            fmasks.append((col >= flo_l) & (col < fhi_l))
        zv = jnp.zeros((_L,), jnp.float32)

        def mask_chunk(buf, r):
            def row_body(j, _):
                t_abs = r + j
                in_band = (t_abs >= tlo) & (t_abs < thi)

                @pl.when(in_band)
                def _():
                    for g in range(_A // _L):
                        buf[j, pl.ds(g * _L, _L)] = zv

                @pl.when(jnp.logical_not(in_band))
                def _():
                    for u in range(n_fg):
                        v = buf[j, pl.ds(fcols[u], _L)]
                        buf[j, pl.ds(fcols[u], _L)] = jnp.where(fmasks[u], 0.0, v)

                return 0

            lax.fori_loop(0, _CH, row_body, 0)

        v_desc().start()
        for s in range(_NBUF):
            in_desc(s, s).start()

        def round_body(i, _):
            c0 = i * _NBUF
            for s in range(_NBUF):
                ci = c0 + s
                in_desc(ci, s).wait()
                mask_chunk(bufs[s], row0 + ci * _CH)
                out_desc(ci, s).start()
            for s in range(_NBUF):
                ci = c0 + s
                nxt = ci + _NBUF

                @pl.when(nxt < n_chunks)
                def _():
                    out_desc(ci, s).wait()
                    in_desc(nxt, s).start()

            return 0

        lax.fori_loop(0, n_rounds, round_body, 0)
        for s in range(_NBUF):
            out_desc(n_chunks - _NBUF + s, s).wait()
        v_desc().wait()

    return k(X, params)


def kernel(X, lengths):
    B, T, D = X.shape
    off = D - _A
    tlo, thi, flo, fhi = _mask_bounds(lengths)
    params = jnp.stack([tlo, thi, flo + off, fhi + off], axis=1)
    params = jnp.broadcast_to(params[:, :, None], (B, 4, _L)).astype(jnp.int32)
    return _sc_call(X, params)


# speedup vs baseline: 11.1621x; 11.1621x over previous
import functools

import jax
import jax.numpy as jnp
from jax import lax
from jax.experimental import pallas as pl
from jax.experimental.pallas import tpu as pltpu
from jax.experimental.pallas import tpu_sc as plsc

_A = 1280
_FR = 0.15
_TR = 0.2

_NC = 2
_NS = 16
_L = 16
_CH = 8
_NBUF = 4


def _mask_bounds(lengths):
    B = lengths.shape[0]
    Ti = lengths.astype(jnp.int32)
    key = jax.random.key(42)
    key, ka, kb = jax.random.split(key, 3)
    max_t = jnp.maximum(1, jnp.floor(Ti.astype(jnp.float32) * _TR).astype(jnp.int32))
    u = jax.random.uniform(ka, (B,))
    t = 1 + jnp.floor(u * max_t.astype(jnp.float32)).astype(jnp.int32)
    t = jnp.minimum(t, max_t)
    room = Ti - t
    u2 = jax.random.uniform(kb, (B,))
    t0 = jnp.where(room > 0,
                   jnp.floor(u2 * (room + 1).astype(jnp.float32)).astype(jnp.int32),
                   0)
    valid = Ti > 0
    big = jnp.int32(1 << 30)
    tlo = jnp.where(valid, t0, big)
    thi = jnp.where(valid, t0 + t, big)
    max_f = int(_A * _FR)
    key, ka2, kb2 = jax.random.split(key, 3)
    f = jax.random.randint(ka2, (B,), 1, max_f + 1)
    f0_max = jnp.clip(_A - f, 0, None)
    f0 = jnp.floor(jax.random.uniform(kb2, (B,)) * (f0_max + 1).astype(jnp.float32)
                   ).astype(jnp.int32)
    return tlo, thi, f0, f0 + f


def _sc_call(X, params):
    B, T, D = X.shape
    off = D - _A
    nw = _NC * _NS
    rows_w = (B * T) // nw
    wps = T // rows_w
    n_chunks = rows_w // _CH
    mesh = plsc.VectorSubcoreMesh(core_axis_name="c", subcore_axis_name="s")

    n_rounds = n_chunks // _NBUF

    @functools.partial(
        pl.kernel,
        mesh=mesh,
        out_type=jax.ShapeDtypeStruct((B, T, D), X.dtype),
        scratch_types=(
            [pltpu.VMEM((_CH, _A), jnp.float32)] * _NBUF
            + [pltpu.VMEM((4, _L), jnp.int32)]
            + [pltpu.VMEM_SHARED((_NS, _NBUF, _CH, D - _A), jnp.float32)]
            + [pltpu.SemaphoreType.DMA] * (4 * _NBUF)
        ),
    )
    def k(x_hbm, p_hbm, out_hbm, *rest):
        bufs = rest[:_NBUF]
        pv = rest[_NBUF]
        vshared = rest[_NBUF + 1]
        sems = rest[_NBUF + 2:]
        isems = sems[:_NBUF]
        osems = sems[_NBUF:2 * _NBUF]
        visems = sems[2 * _NBUF:3 * _NBUF]
        vosems = sems[3 * _NBUF:]
        cid = lax.axis_index("c")
        sid = lax.axis_index("s")
        wid = sid * _NC + cid
        b = wid // wps
        row0 = (wid % wps) * rows_w
        pltpu.sync_copy(p_hbm.at[b], pv)
        tlo = pv[0][0]
        thi = pv[1][0]
        flo = pv[2][0]
        fhi = pv[3][0]

        def in_desc(ci, s):
            r = row0 + ci * _CH
            return pltpu.make_async_copy(
                x_hbm.at[b, pl.ds(r, _CH), pl.ds(off, _A)], bufs[s], isems[s])

        def out_desc(ci, s):
            r = row0 + ci * _CH
            return pltpu.make_async_copy(
                bufs[s], out_hbm.at[b, pl.ds(r, _CH), pl.ds(off, _A)], osems[s])

        def vin_desc(ci, s):
            r = row0 + ci * _CH
            return pltpu.make_async_copy(
                x_hbm.at[b, pl.ds(r, _CH), pl.ds(0, off)],
                vshared.at[sid, s], visems[s])

        def vout_desc(ci, s):
            r = row0 + ci * _CH
            return pltpu.make_async_copy(
                vshared.at[sid, s],
                out_hbm.at[b, pl.ds(r, _CH), pl.ds(0, off)], vosems[s])

        flo_l = flo - off
        fhi_l = fhi - off
        n_fg = int(_A * _FR) // _L + 2
        fg0 = flo_l // _L
        lanes = lax.iota(jnp.int32, _L)
        fcols = []
        fmasks = []
        for u in range(n_fg):
            g = jnp.minimum(fg0 + u, (_A // _L) - 1)
            c0 = g * _L
            col = c0 + lanes
            fcols.append(c0)
            fmasks.append((col >= flo_l) & (col < fhi_l))
        zv = jnp.zeros((_L,), jnp.float32)

        def mask_chunk(buf, r):
            def row_body(j, _):
                t_abs = r + j
                in_band = (t_abs >= tlo) & (t_abs < thi)

                @pl.when(in_band)
                def _():
                    for g in range(_A // _L):
                        buf[j, pl.ds(g * _L, _L)] = zv

                @pl.when(jnp.logical_not(in_band))
                def _():
                    for u in range(n_fg):
                        v = buf[j, pl.ds(fcols[u], _L)]
                        buf[j, pl.ds(fcols[u], _L)] = jnp.where(fmasks[u], 0.0, v)

                return 0

            lax.fori_loop(0, _CH, row_body, 0)

        for s in range(_NBUF):
            in_desc(s, s).start()
            vin_desc(s, s).start()

        def round_body(i, _):
            c0 = i * _NBUF
            for s in range(_NBUF):
                ci = c0 + s
                vin_desc(ci, s).wait()
                vout_desc(ci, s).start()
                in_desc(ci, s).wait()
                mask_chunk(bufs[s], row0 + ci * _CH)
                out_desc(ci, s).start()
            for s in range(_NBUF):
                ci = c0 + s
                nxt = ci + _NBUF

                @pl.when(nxt < n_chunks)
                def _():
                    out_desc(ci, s).wait()
                    in_desc(nxt, s).start()
                    vout_desc(ci, s).wait()
                    vin_desc(nxt, s).start()

            return 0

        lax.fori_loop(0, n_rounds, round_body, 0)
        for s in range(_NBUF):
            out_desc(n_chunks - _NBUF + s, s).wait()
            vout_desc(n_chunks - _NBUF + s, s).wait()

    return k(X, params)


def kernel(X, lengths):
    B, T, D = X.shape
    off = D - _A
    tlo, thi, flo, fhi = _mask_bounds(lengths)
    params = jnp.stack([tlo, thi, flo + off, fhi + off], axis=1)
    params = jnp.broadcast_to(params[:, :, None], (B, 4, _L)).astype(jnp.int32)
    return _sc_call(X, params)
